# baseline (speedup 1.0000x reference)
import functools

import jax
import jax.numpy as jnp
from jax import lax
from jax.experimental import pallas as pl
from jax.experimental.pallas import tpu as pltpu
from jax.experimental.pallas import tpu_sc as plsc

NUM_CORES = 2
NUM_SUBCORES = 16
NW = NUM_CORES * NUM_SUBCORES
LANES = 16
DIM = 64
TOTAL = 2 * 16384
B_PER_W = TOTAL // NW
CHUNK = 128
NCHUNK = B_PER_W // CHUNK
BLOCKS = CHUNK // LANES


def _sqrt_f32(x):
    i = plsc.bitcast(x, jnp.int32)
    y = plsc.bitcast(jnp.int32(0x5F3759DF) - (i >> 1), jnp.float32)
    for _ in range(3):
        y = y * (1.5 - 0.5 * x * y * y)
    return x * y


def _transe_body(ent_hbm, rel_hbm, h_hbm, r_hbm, t_hbm, out_hbm,
                 hidx, ridx, tidx, hrows, rrows, trows, sums, sem):
    wid = lax.axis_index("s") * NUM_CORES + lax.axis_index("c")
    base = wid * B_PER_W

    idx_copies = []
    for ci in range(NCHUNK):
        src = pl.ds(base + ci * CHUNK, CHUNK)
        idx_copies.append(pltpu.async_copy(h_hbm.at[src], hidx.at[ci], sem))
        idx_copies.append(pltpu.async_copy(r_hbm.at[src], ridx.at[ci], sem))
        idx_copies.append(pltpu.async_copy(t_hbm.at[src], tidx.at[ci], sem))
    for c in idx_copies:
        c.wait()

    lane_iota = lax.iota(jnp.int32, LANES)

    def chunk_body(ci, carry):
        ch = pltpu.async_copy(ent_hbm.at[hidx.at[ci]], hrows, sem)
        cr = pltpu.async_copy(rel_hbm.at[ridx.at[ci]], rrows, sem)
        ct = pltpu.async_copy(ent_hbm.at[tidx.at[ci]], trows, sem)
        ch.wait()
        cr.wait()
        ct.wait()

        def blk_body(b, carry):
            rowv = b * LANES + lane_iota
            acc = jnp.zeros((LANES,), jnp.float32)
            for j in range(DIM):
                cj = jnp.full((LANES,), j, jnp.int32)
                hv = plsc.load_gather(hrows, [rowv, cj])
                rv = plsc.load_gather(rrows, [rowv, cj])
                tv = plsc.load_gather(trows, [rowv, cj])
                d = hv + rv - tv
                acc = acc + d * d
            sums[pl.ds(ci * CHUNK + b * LANES, LANES)] = _sqrt_f32(acc)
            return carry

        return lax.fori_loop(0, BLOCKS, blk_body, carry)

    lax.fori_loop(0, NCHUNK, chunk_body, 0)
    pltpu.sync_copy(sums, out_hbm.at[pl.ds(base, B_PER_W)])


@jax.jit
def kernel(entity_embeddings, relation_embeddings,
           positive_head_batch, positive_relation_batch, positive_tail_batch,
           negative_head_batch, negative_relation_batch, negative_tail_batch):
    heads = jnp.concatenate([positive_head_batch, negative_head_batch]).astype(jnp.int32)
    rels = jnp.concatenate([positive_relation_batch, negative_relation_batch]).astype(jnp.int32)
    tails = jnp.concatenate([positive_tail_batch, negative_tail_batch]).astype(jnp.int32)

    k = pl.kernel(
        _transe_body,
        out_type=jax.ShapeDtypeStruct((TOTAL,), jnp.float32),
        mesh=plsc.VectorSubcoreMesh(core_axis_name="c", subcore_axis_name="s"),
        scratch_types=[
            pltpu.VMEM((NCHUNK, CHUNK), jnp.int32),
            pltpu.VMEM((NCHUNK, CHUNK), jnp.int32),
            pltpu.VMEM((NCHUNK, CHUNK), jnp.int32),
            pltpu.VMEM((CHUNK, DIM), jnp.float32),
            pltpu.VMEM((CHUNK, DIM), jnp.float32),
            pltpu.VMEM((CHUNK, DIM), jnp.float32),
            pltpu.VMEM((B_PER_W,), jnp.float32),
            pltpu.SemaphoreType.DMA,
        ],
        compiler_params=pltpu.CompilerParams(
            needs_layout_passes=False, use_tc_tiling_on_sc=False),
        name="transe_sc",
    )
    losses = k(entity_embeddings, relation_embeddings, heads, rels, tails)
    return losses.reshape(2, 16384)


# speedup vs baseline: 1.0102x; 1.0102x over previous
import jax
import jax.numpy as jnp
from jax import lax
from jax.experimental import pallas as pl
from jax.experimental.pallas import tpu as pltpu
from jax.experimental.pallas import tpu_sc as plsc

NUM_CORES = 2
NUM_SUBCORES = 16
NW = NUM_CORES * NUM_SUBCORES
LANES = 16
DIM = 64
TOTAL = 2 * 16384
B_PER_W = TOTAL // NW
IDX_ROW = 128
NIDX = B_PER_W // IDX_ROW
CHUNK = 256
NCHUNK = B_PER_W // CHUNK
GPC = CHUNK // IDX_ROW
BLOCKS = CHUNK // LANES


def _sqrt_f32(x):
    i = plsc.bitcast(x, jnp.int32)
    y = plsc.bitcast(jnp.int32(0x5F3759DF) - (i >> 1), jnp.float32)
    for _ in range(3):
        y = y * (1.5 - 0.5 * x * y * y)
    return x * y


def _transe_body(ent_hbm, rel_hbm, h_hbm, r_hbm, t_hbm, out_hbm,
                 hidx, ridx, tidx,
                 hrows0, hrows1, rrows0, rrows1, trows0, trows1,
                 sums, sem):
    wid = lax.axis_index("s") * NUM_CORES + lax.axis_index("c")

    ih = pltpu.async_copy(h_hbm.at[wid], hidx, sem)
    ir = pltpu.async_copy(r_hbm.at[wid], ridx, sem)
    it = pltpu.async_copy(t_hbm.at[wid], tidx, sem)
    ih.wait()
    ir.wait()
    it.wait()

    hbufs = (hrows0, hrows1)
    rbufs = (rrows0, rrows1)
    tbufs = (trows0, trows1)

    def fire(c):
        hb, rb, tb = hbufs[c % 2], rbufs[c % 2], tbufs[c % 2]
        cps = []
        for g in range(GPC):
            j = c * GPC + g
            dst = pl.ds(g * IDX_ROW, IDX_ROW)
            cps.append(pltpu.async_copy(ent_hbm.at[hidx.at[j]], hb.at[dst], sem))
            cps.append(pltpu.async_copy(rel_hbm.at[ridx.at[j]], rb.at[dst], sem))
            cps.append(pltpu.async_copy(ent_hbm.at[tidx.at[j]], tb.at[dst], sem))
        return cps

    lane_iota = lax.iota(jnp.int32, LANES)

    def compute(c):
        hb, rb, tb = hbufs[c % 2], rbufs[c % 2], tbufs[c % 2]

        def blk_body(b, carry):
            rowv = b * LANES + lane_iota
            acc = jnp.zeros((LANES,), jnp.float32)
            for j in range(DIM):
                cj = jnp.full((LANES,), j, jnp.int32)
                hv = plsc.load_gather(hb, [rowv, cj])
                rv = plsc.load_gather(rb, [rowv, cj])
                tv = plsc.load_gather(tb, [rowv, cj])
                d = hv + rv - tv
                acc = acc + d * d
            sums[pl.ds(c * CHUNK + b * LANES, LANES)] = _sqrt_f32(acc)
            return carry

        lax.fori_loop(0, BLOCKS, blk_body, 0)

    inflight = fire(0)
    for c in range(NCHUNK):
        nxt = fire(c + 1) if c + 1 < NCHUNK else []
        for cp in inflight:
            cp.wait()
        compute(c)
        inflight = nxt

    pltpu.sync_copy(sums, out_hbm.at[pl.ds(wid * B_PER_W, B_PER_W)])


@jax.jit
def kernel(entity_embeddings, relation_embeddings,
           positive_head_batch, positive_relation_batch, positive_tail_batch,
           negative_head_batch, negative_relation_batch, negative_tail_batch):
    heads = jnp.concatenate([positive_head_batch, negative_head_batch])
    rels = jnp.concatenate([positive_relation_batch, negative_relation_batch])
    tails = jnp.concatenate([positive_tail_batch, negative_tail_batch])
    heads = heads.astype(jnp.int32).reshape(NW, NIDX, IDX_ROW)
    rels = rels.astype(jnp.int32).reshape(NW, NIDX, IDX_ROW)
    tails = tails.astype(jnp.int32).reshape(NW, NIDX, IDX_ROW)

    k = pl.kernel(
        _transe_body,
        out_type=jax.ShapeDtypeStruct((TOTAL,), jnp.float32),
        mesh=plsc.VectorSubcoreMesh(core_axis_name="c", subcore_axis_name="s"),
        scratch_types=[
            pltpu.VMEM((NIDX, IDX_ROW), jnp.int32),
            pltpu.VMEM((NIDX, IDX_ROW), jnp.int32),
            pltpu.VMEM((NIDX, IDX_ROW), jnp.int32),
            pltpu.VMEM((CHUNK, DIM), jnp.float32),
            pltpu.VMEM((CHUNK, DIM), jnp.float32),
            pltpu.VMEM((CHUNK, DIM), jnp.float32),
            pltpu.VMEM((CHUNK, DIM), jnp.float32),
            pltpu.VMEM((CHUNK, DIM), jnp.float32),
            pltpu.VMEM((CHUNK, DIM), jnp.float32),
            pltpu.VMEM((B_PER_W,), jnp.float32),
            pltpu.SemaphoreType.DMA,
        ],
        compiler_params=pltpu.CompilerParams(
            needs_layout_passes=False, use_tc_tiling_on_sc=False),
        name="transe_sc",
    )
    losses = k(entity_embeddings, relation_embeddings, heads, rels, tails)
    return losses.reshape(2, 16384)
